# Initial kernel scaffold; baseline (speedup 1.0000x reference)
import jax
import jax.numpy as jnp
from jax.experimental import pallas as pl


def kernel(outputs, target_boxes, target_keypoints):
    raise NotImplementedError("write your pallas kernel here")


# speedup vs baseline: 5.2477x; 5.2477x over previous
import functools

import jax
import jax.numpy as jnp
from jax.experimental import pallas as pl

_G = 32
_ANCHOR = 2.5
_EMPTY_WEIGHT = 0.5


def _loss_kernel(tb_ref, tk_ref, out_ref, acc_ref):
    tb = tb_ref[0]
    tk = tk_ref[0]
    grid_out = out_ref[0]

    txy = tb[:, 0:2] * float(_G)
    tx_full = txy[:, 0:1]
    ty_full = txy[:, 1:2]
    gi = jnp.clip(tx_full.astype(jnp.int32), 0, _G - 1)
    gj = jnp.clip(ty_full.astype(jnp.int32), 0, _G - 1)
    cell = gj * _G + gi

    n_obj = cell.shape[0]
    eq = cell == cell.T
    row_i = jax.lax.broadcasted_iota(jnp.int32, (n_obj, n_obj), 0)
    col_j = jax.lax.broadcasted_iota(jnp.int32, (n_obj, n_obj), 1)
    later_same = jnp.logical_and(eq, col_j > row_i)
    loses = jnp.any(later_same, axis=1, keepdims=True)
    winner = jnp.logical_not(loses).astype(jnp.float32)

    lane = jax.lax.broadcasted_iota(jnp.int32, (n_obj, _G * _G), 1)
    onehot = (cell == lane).astype(jnp.float32)
    gathered = jnp.dot(onehot, grid_out,
                       preferred_element_type=jnp.float32)

    x = gathered[:, 0:1]
    y = gathered[:, 1:2]
    w = gathered[:, 2:3]
    h = gathered[:, 3:4]
    conf = gathered[:, 4:5]
    keyp = gathered[:, 5:68]

    fx = tx_full - jnp.floor(tx_full)
    fy = ty_full - jnp.floor(ty_full)
    tw = tb[:, 3:4] * (float(_G) / _ANCHOR)
    th = tb[:, 4:5] * (float(_G) / _ANCHOR)

    sx = jnp.sum(winner * (x - fx) ** 2)
    sy = jnp.sum(winner * (y - fy) ** 2)
    sw = jnp.sum(winner * (w - tw) ** 2)
    sh = jnp.sum(winner * (h - th) ** 2)

    logp = jnp.maximum(jnp.log(conf), -100.0)
    sconf = jnp.sum(winner * (_EMPTY_WEIGHT * (-logp)))

    skp = jnp.sum(winner * jnp.abs(keyp - tk))

    b1x1, b1x2 = x - w * 0.5, x + w * 0.5
    b1y1, b1y2 = y - h * 0.5, y + h * 0.5
    b2x1, b2x2 = fx - tw * 0.5, fx + tw * 0.5
    b2y1, b2y2 = fy - th * 0.5, fy + th * 0.5
    iw = jnp.maximum(jnp.minimum(b1x2, b2x2) - jnp.maximum(b1x1, b2x1), 0.0)
    ih = jnp.maximum(jnp.minimum(b1y2, b2y2) - jnp.maximum(b1y1, b2y1), 0.0)
    inter = iw * ih
    union = w * h + tw * th - inter + 1e-16
    iou = inter / union
    siou = jnp.sum(winner * (1.0 - iou))

    n_here = jnp.sum(winner)

    vals = jnp.stack([sx, sy, sw, sh, sconf, skp, siou, n_here])
    vals = vals.reshape(1, 8)

    @pl.when(pl.program_id(0) == 0)
    def _init():
        acc_ref[...] = jnp.zeros_like(acc_ref)

    acc_ref[...] += vals


@jax.jit
def kernel(outputs, target_boxes, target_keypoints):
    B, G = outputs.shape[0], outputs.shape[1]
    out_flat = outputs.reshape(B, G * G, outputs.shape[3])

    sums = pl.pallas_call(
        _loss_kernel,
        grid=(B,),
        in_specs=[
            pl.BlockSpec((1, 50, 6), lambda b: (b, 0, 0)),
            pl.BlockSpec((1, 50, 63), lambda b: (b, 0, 0)),
            pl.BlockSpec((1, G * G, outputs.shape[3]), lambda b: (b, 0, 0)),
        ],
        out_specs=pl.BlockSpec((1, 8), lambda b: (0, 0)),
        out_shape=jax.ShapeDtypeStruct((1, 8), jnp.float32),
    )(target_boxes, target_keypoints, out_flat)

    s = sums[0]
    n = s[7]
    loss_boxes = (s[0] + s[1] + s[2] + s[3]) / n
    loss_keypoint = s[5] / n
    loss_conf = s[4] / (n * n)
    loss_iou = s[6] / n
    return (loss_boxes, loss_keypoint, loss_conf, loss_iou)
